# Initial kernel scaffold; baseline (speedup 1.0000x reference)
import jax
import jax.numpy as jnp
from jax.experimental import pallas as pl


def kernel(edge_index, edge_weight):
    raise NotImplementedError("write your pallas kernel here")


# speedup vs baseline: 401.1346x; 401.1346x over previous
import functools

import jax
import jax.numpy as jnp
from jax import lax
from jax.experimental import pallas as pl
from jax.experimental.pallas import tpu as pltpu
from jax.experimental.pallas import tpu_sc as plsc

N_NODES = 100_000
N_EDGES = 6_400_000

NC = 2
NS = 16
NW = NC * NS
L = 16

P = 100_352
SLICE = P // NS
WSLICE = P // NW
NTAIL = 25
TS = N_NODES // NTAIL

C1 = 6_400
B1 = C1 // 128
NCH1 = N_EDGES // C1
MX1 = 33

C3 = 2_560
B3 = C3 // 128
NCH3 = N_EDGES // C3
MX3 = 80

assert NCH1 * C1 == N_EDGES and C1 % 128 == 0 and MX1 % 3 == 0
assert NCH3 * C3 == N_EDGES and C3 % 128 == 0 and MX3 % 2 == 0
assert MX1 >= -(-NCH1 // NW) and MX3 >= -(-NCH3 // NW)
assert P % (NS * L) == 0 and P % (NW * L) == 0 and P >= N_NODES
assert NTAIL * TS == N_NODES and TS % L == 0

_mesh = functools.partial(
    plsc.VectorSubcoreMesh,
    core_axis_name="c", subcore_axis_name="s", num_cores=NC, num_subcores=NS,
)


def _wid():
    return lax.axis_index("c") * NS + lax.axis_index("s")


def _deg_body(ei_hbm, ew_hbm, deg_hbm,
              ebuf0, ebuf1, ebuf2, wbuf0, wbuf1, wbuf2,
              rbuf0, rbuf1, rbuf2, zrow, deg_sp,
              isem0, isem1, isem2, ssem0, ssem1, ssem2):
    c = lax.axis_index("c")
    s = lax.axis_index("s")
    wid = c * NS + s
    ebufs = (ebuf0, ebuf1, ebuf2)
    wbufs = (wbuf0, wbuf1, wbuf2)
    rbufs = (rbuf0, rbuf1, rbuf2)
    isems = (isem0, isem1, isem2)
    ssems = (ssem0, ssem1, ssem2)

    zeros16 = jnp.zeros((L,), jnp.float32)

    def _zero(i, carry):
        zrow[pl.ds(i * L, L)] = zeros16
        return carry

    lax.fori_loop(0, SLICE // L, _zero, None, unroll=8)
    pltpu.sync_copy(zrow, deg_sp.at[pl.ds(s * SLICE, SLICE)])

    def _issue_in(k, b):
        pltpu.async_copy(ei_hbm.at[pl.ds(k * B1, B1)], ebufs[b], isems[b])
        pltpu.async_copy(ew_hbm.at[pl.ds(k * C1, C1)], wbufs[b], isems[b])

    def _drain_scatter(b):
        pltpu.make_async_copy(wbufs[b], deg_sp.at[rbufs[b]], ssems[b]).wait()

    _issue_in(wid, 0)
    plsc.subcore_barrier()

    def _trip(j3, carry):
        for b in range(3):
            j = j3 * 3 + b
            k = wid + j * NW
            nslot = (b + 1) % 3

            @pl.when((k - 2 * NW >= 0) & (k - 2 * NW < NCH1) & (k - 2 * NW >= wid))
            def _():
                _drain_scatter(nslot)

            @pl.when(k + NW < NCH1)
            def _():
                _issue_in(k + NW, nslot)

            @pl.when(k < NCH1)
            def _():
                pltpu.make_async_copy(
                    ei_hbm.at[pl.ds(k * B1, B1)], ebufs[b], isems[b]).wait()
                pltpu.make_async_copy(
                    ew_hbm.at[pl.ds(k * C1, C1)], wbufs[b], isems[b]).wait()

                def _pack(i, carry2):
                    blk = lax.shift_right_logical(i, 3)
                    o = (i & 7) * L
                    rbufs[b][pl.ds(i * L, L)] = ebufs[b][blk, 0, pl.ds(o, L)]
                    return carry2

                lax.fori_loop(0, C1 // L, _pack, None, unroll=8)
                pltpu.async_copy(wbufs[b], deg_sp.at[rbufs[b]], ssems[b],
                                 add=True)

        return carry

    lax.fori_loop(0, MX1 // 3, _trip, None)

    for j in (MX1 - 2, MX1 - 1):
        @pl.when(wid + j * NW < NCH1)
        def _():
            _drain_scatter(j % 3)
    plsc.subcore_barrier()
    pltpu.sync_copy(deg_sp.at[pl.ds(s * SLICE, SLICE)],
                    deg_hbm.at[pl.ds(c * P + s * SLICE, SLICE)])


def _rsqrt_body(deg_hbm, dinv_hbm, b0, b1):
    base = _wid() * WSLICE
    pltpu.sync_copy(deg_hbm.at[pl.ds(base, WSLICE)], b0)
    pltpu.sync_copy(deg_hbm.at[pl.ds(P + base, WSLICE)], b1)

    def _it(i, carry):
        sl = pl.ds(i * L, L)
        d = b0[sl] + b1[sl] + 1.0
        bits = lax.bitcast_convert_type(d, jnp.int32)
        bits = 0x5F3759DF - lax.shift_right_arithmetic(bits, 1)
        y = lax.bitcast_convert_type(bits, jnp.float32)
        xh = d * 0.5
        y = y * (1.5 - xh * y * y)
        y = y * (1.5 - xh * y * y)
        y = y * (1.5 - xh * y * y)
        b0[sl] = y
        return carry

    lax.fori_loop(0, WSLICE // L, _it, None, unroll=8)
    pltpu.sync_copy(b0, dinv_hbm.at[pl.ds(base, WSLICE)])


def _norm_body(ei_hbm, ew_hbm, dinv_hbm, out_hbm,
               dv, ebuf0, ebuf1, wbuf0, wbuf1, obuf0, obuf1, tbuf,
               bsem, isem0, isem1, osem0, osem1):
    wid = _wid()
    ebufs, wbufs, obufs = (ebuf0, ebuf1), (wbuf0, wbuf1), (obuf0, obuf1)
    isems, osems = (isem0, isem1), (osem0, osem1)

    def _issue_in(k, b):
        pltpu.async_copy(ei_hbm.at[pl.ds(k * B3, B3)], ebufs[b], isems[b])
        pltpu.async_copy(ew_hbm.at[pl.ds(k * C3, C3)], wbufs[b], isems[b])

    _issue_in(wid, 0)

    for i in range(NW):
        p = (wid + i) % NW
        pltpu.async_copy(dinv_hbm.at[pl.ds(p * WSLICE, WSLICE)],
                         dv.at[pl.ds(p * WSLICE, WSLICE)], bsem)
    for i in range(NW):
        p = (wid + i) % NW
        pltpu.make_async_copy(dinv_hbm.at[pl.ds(p * WSLICE, WSLICE)],
                              dv.at[pl.ds(p * WSLICE, WSLICE)], bsem).wait()

    def _pair(j2, carry):
        for b in range(2):
            j = j2 * 2 + b
            k = wid + j * NW
            knext = k + NW

            @pl.when(knext < NCH3)
            def _():
                _issue_in(knext, 1 - b)

            @pl.when(k < NCH3)
            def _():
                pltpu.make_async_copy(
                    ei_hbm.at[pl.ds(k * B3, B3)], ebufs[b], isems[b]).wait()
                pltpu.make_async_copy(
                    ew_hbm.at[pl.ds(k * C3, C3)], wbufs[b], isems[b]).wait()

                @pl.when(j >= 2)
                def _():
                    pltpu.make_async_copy(
                        obufs[b], out_hbm.at[pl.ds(k * C3, C3)],
                        osems[b]).wait()

                def _inner(i, carry2):
                    blk = lax.shift_right_logical(i, 3)
                    o = (i & 7) * L
                    r = ebufs[b][blk, 0, pl.ds(o, L)]
                    cl = ebufs[b][blk, 1, pl.ds(o, L)]
                    a = plsc.load_gather(dv, [r])
                    bb = plsc.load_gather(dv, [cl])
                    sl = pl.ds(i * L, L)
                    obufs[b][sl] = a * wbufs[b][sl] * bb
                    return carry2

                lax.fori_loop(0, C3 // L, _inner, None, unroll=8)
                pltpu.async_copy(obufs[b], out_hbm.at[pl.ds(k * C3, C3)],
                                 osems[b])

        return carry

    lax.fori_loop(0, MX3 // 2, _pair, None)

    @pl.when(wid < NTAIL)
    def _tail():
        tbase = wid * TS

        def _it(i, carry):
            v = dv[pl.ds(tbase + i * L, L)]
            tbuf[pl.ds(i * L, L)] = v * v
            return carry

        lax.fori_loop(0, TS // L, _it, None, unroll=8)
        pltpu.sync_copy(tbuf, out_hbm.at[pl.ds(N_EDGES + tbase, TS)])

    for b in range(2):
        pltpu.make_async_copy(obufs[b], out_hbm.at[pl.ds(0, C3)],
                              osems[b]).wait()


_deg_call = pl.kernel(
    _deg_body,
    out_type=jax.ShapeDtypeStruct((NC * P,), jnp.float32),
    mesh=_mesh(),
    compiler_params=pltpu.CompilerParams(needs_layout_passes=False),
    scratch_types=[
        pltpu.VMEM((B1, 2, 128), jnp.int32),
        pltpu.VMEM((B1, 2, 128), jnp.int32),
        pltpu.VMEM((B1, 2, 128), jnp.int32),
        pltpu.VMEM((C1,), jnp.float32),
        pltpu.VMEM((C1,), jnp.float32),
        pltpu.VMEM((C1,), jnp.float32),
        pltpu.VMEM((C1,), jnp.int32),
        pltpu.VMEM((C1,), jnp.int32),
        pltpu.VMEM((C1,), jnp.int32),
        pltpu.VMEM((SLICE,), jnp.float32),
        pltpu.VMEM_SHARED((P,), jnp.float32),
        pltpu.SemaphoreType.DMA,
        pltpu.SemaphoreType.DMA,
        pltpu.SemaphoreType.DMA,
        pltpu.SemaphoreType.DMA,
        pltpu.SemaphoreType.DMA,
        pltpu.SemaphoreType.DMA,
    ],
)

_rsqrt_call = pl.kernel(
    _rsqrt_body,
    out_type=jax.ShapeDtypeStruct((P,), jnp.float32),
    mesh=_mesh(),
    compiler_params=pltpu.CompilerParams(needs_layout_passes=False),
    scratch_types=[
        pltpu.VMEM((WSLICE,), jnp.float32),
        pltpu.VMEM((WSLICE,), jnp.float32),
    ],
)

_norm_call = pl.kernel(
    _norm_body,
    out_type=jax.ShapeDtypeStruct((N_EDGES + N_NODES,), jnp.float32),
    mesh=_mesh(),
    compiler_params=pltpu.CompilerParams(needs_layout_passes=False),
    scratch_types=[
        pltpu.VMEM((P,), jnp.float32),
        pltpu.VMEM((B3, 2, 128), jnp.int32),
        pltpu.VMEM((B3, 2, 128), jnp.int32),
        pltpu.VMEM((C3,), jnp.float32),
        pltpu.VMEM((C3,), jnp.float32),
        pltpu.VMEM((C3,), jnp.float32),
        pltpu.VMEM((C3,), jnp.float32),
        pltpu.VMEM((TS,), jnp.float32),
        pltpu.SemaphoreType.DMA,
        pltpu.SemaphoreType.DMA,
        pltpu.SemaphoreType.DMA,
        pltpu.SemaphoreType.DMA,
        pltpu.SemaphoreType.DMA,
    ],
)


def kernel(edge_index, edge_weight):
    ei3 = edge_index.reshape(N_EDGES // 128, 128, 2).transpose(0, 2, 1)
    deg2 = _deg_call(ei3, edge_weight)
    dinv = _rsqrt_call(deg2)
    normed = _norm_call(ei3, edge_weight, dinv)
    ar = jnp.arange(N_NODES, dtype=edge_index.dtype)
    diag = jnp.stack([ar, ar], axis=1)
    ei = jnp.concatenate([edge_index, diag], axis=0)
    return ei, normed


# speedup vs baseline: 449.9668x; 1.1217x over previous
import functools

import jax
import jax.numpy as jnp
from jax import lax
from jax.experimental import pallas as pl
from jax.experimental.pallas import tpu as pltpu
from jax.experimental.pallas import tpu_sc as plsc

N_NODES = 100_000
N_EDGES = 6_400_000

NC = 2
NS = 16
NW = NC * NS
L = 16

P = 100_352
SLICE = P // NS
WSLICE = P // NW
NTAIL = 25
TS = N_NODES // NTAIL

C1 = 6_400
B1 = C1 // 128
NCH1 = N_EDGES // C1
MX1 = 33

C3 = 2_560
B3 = C3 // 128
NCH3 = N_EDGES // C3
MX3 = 80

assert NCH1 * C1 == N_EDGES and C1 % 128 == 0 and MX1 % 3 == 0
assert NCH3 * C3 == N_EDGES and C3 % 128 == 0 and MX3 % 2 == 0
assert MX1 >= -(-NCH1 // NW) and MX3 >= -(-NCH3 // NW)
assert P % (NS * L) == 0 and P % (NW * L) == 0 and P >= N_NODES
assert NTAIL * TS == N_NODES and TS % L == 0

_mesh = functools.partial(
    plsc.VectorSubcoreMesh,
    core_axis_name="c", subcore_axis_name="s", num_cores=NC, num_subcores=NS,
)


def _wid():
    return lax.axis_index("c") * NS + lax.axis_index("s")


def _deg_body(ei_hbm, ew_hbm, deg_hbm,
              ebuf0, ebuf1, ebuf2, wbuf0, wbuf1, wbuf2,
              rbuf0, rbuf1, rbuf2, zrow, deg_sp,
              isem0, isem1, isem2, ssem0, ssem1, ssem2):
    c = lax.axis_index("c")
    s = lax.axis_index("s")
    wid = c * NS + s
    ebufs = (ebuf0, ebuf1, ebuf2)
    wbufs = (wbuf0, wbuf1, wbuf2)
    rbufs = (rbuf0, rbuf1, rbuf2)
    isems = (isem0, isem1, isem2)
    ssems = (ssem0, ssem1, ssem2)

    zeros16 = jnp.zeros((L,), jnp.float32)

    def _zero(i, carry):
        zrow[pl.ds(i * L, L)] = zeros16
        return carry

    lax.fori_loop(0, SLICE // L, _zero, None, unroll=8)
    pltpu.sync_copy(zrow, deg_sp.at[pl.ds(s * SLICE, SLICE)])

    def _issue_in(k, b):
        pltpu.async_copy(ei_hbm.at[pl.ds(k * B1, B1)], ebufs[b], isems[b])
        pltpu.async_copy(ew_hbm.at[pl.ds(k * C1, C1)], wbufs[b], isems[b])

    def _drain_scatter(b):
        pltpu.make_async_copy(wbufs[b], deg_sp.at[rbufs[b]], ssems[b]).wait()

    _issue_in(wid, 0)
    plsc.subcore_barrier()

    def _trip(j3, carry):
        for b in range(3):
            j = j3 * 3 + b
            k = wid + j * NW
            nslot = (b + 1) % 3

            @pl.when((k - 2 * NW >= 0) & (k - 2 * NW < NCH1) & (k - 2 * NW >= wid))
            def _():
                _drain_scatter(nslot)

            @pl.when(k + NW < NCH1)
            def _():
                _issue_in(k + NW, nslot)

            @pl.when(k < NCH1)
            def _():
                pltpu.make_async_copy(
                    ei_hbm.at[pl.ds(k * B1, B1)], ebufs[b], isems[b]).wait()
                pltpu.make_async_copy(
                    ew_hbm.at[pl.ds(k * C1, C1)], wbufs[b], isems[b]).wait()

                def _pack(i, carry2):
                    vs = [ebufs[b][i, 0, pl.ds(o * L, L)] for o in range(8)]
                    for o in range(8):
                        rbufs[b][pl.ds(i * 128 + o * L, L)] = vs[o]
                    return carry2

                lax.fori_loop(0, B1, _pack, None)
                pltpu.async_copy(wbufs[b], deg_sp.at[rbufs[b]], ssems[b],
                                 add=True)

        return carry

    lax.fori_loop(0, MX1 // 3, _trip, None)

    for j in (MX1 - 2, MX1 - 1):
        @pl.when(wid + j * NW < NCH1)
        def _():
            _drain_scatter(j % 3)
    plsc.subcore_barrier()
    pltpu.sync_copy(deg_sp.at[pl.ds(s * SLICE, SLICE)],
                    deg_hbm.at[pl.ds(c * P + s * SLICE, SLICE)])


def _rsqrt_body(deg_hbm, dinv_hbm, b0, b1):
    base = _wid() * WSLICE
    pltpu.sync_copy(deg_hbm.at[pl.ds(base, WSLICE)], b0)
    pltpu.sync_copy(deg_hbm.at[pl.ds(P + base, WSLICE)], b1)

    def _it(i, carry):
        sl = pl.ds(i * L, L)
        d = b0[sl] + b1[sl] + 1.0
        bits = lax.bitcast_convert_type(d, jnp.int32)
        bits = 0x5F3759DF - lax.shift_right_arithmetic(bits, 1)
        y = lax.bitcast_convert_type(bits, jnp.float32)
        xh = d * 0.5
        y = y * (1.5 - xh * y * y)
        y = y * (1.5 - xh * y * y)
        y = y * (1.5 - xh * y * y)
        b0[sl] = y
        return carry

    lax.fori_loop(0, WSLICE // L, _it, None, unroll=8)
    pltpu.sync_copy(b0, dinv_hbm.at[pl.ds(base, WSLICE)])


def _norm_body(ei_hbm, ew_hbm, dinv_hbm, out_hbm,
               dv, ebuf0, ebuf1, wbuf0, wbuf1, obuf0, obuf1, tbuf,
               bsem, isem0, isem1, osem0, osem1):
    wid = _wid()
    ebufs, wbufs, obufs = (ebuf0, ebuf1), (wbuf0, wbuf1), (obuf0, obuf1)
    isems, osems = (isem0, isem1), (osem0, osem1)

    def _issue_in(k, b):
        pltpu.async_copy(ei_hbm.at[pl.ds(k * B3, B3)], ebufs[b], isems[b])
        pltpu.async_copy(ew_hbm.at[pl.ds(k * C3, C3)], wbufs[b], isems[b])

    _issue_in(wid, 0)

    for i in range(NW):
        p = (wid + i) % NW
        pltpu.async_copy(dinv_hbm.at[pl.ds(p * WSLICE, WSLICE)],
                         dv.at[pl.ds(p * WSLICE, WSLICE)], bsem)
    for i in range(NW):
        p = (wid + i) % NW
        pltpu.make_async_copy(dinv_hbm.at[pl.ds(p * WSLICE, WSLICE)],
                              dv.at[pl.ds(p * WSLICE, WSLICE)], bsem).wait()

    def _pair(j2, carry):
        for b in range(2):
            j = j2 * 2 + b
            k = wid + j * NW
            knext = k + NW

            @pl.when(knext < NCH3)
            def _():
                _issue_in(knext, 1 - b)

            @pl.when(k < NCH3)
            def _():
                pltpu.make_async_copy(
                    ei_hbm.at[pl.ds(k * B3, B3)], ebufs[b], isems[b]).wait()
                pltpu.make_async_copy(
                    ew_hbm.at[pl.ds(k * C3, C3)], wbufs[b], isems[b]).wait()

                @pl.when(j >= 2)
                def _():
                    pltpu.make_async_copy(
                        obufs[b], out_hbm.at[pl.ds(k * C3, C3)],
                        osems[b]).wait()

                def _inner(i, carry2):
                    rs = [ebufs[b][i, 0, pl.ds(o * L, L)] for o in range(8)]
                    cs = [ebufs[b][i, 1, pl.ds(o * L, L)] for o in range(8)]
                    ws = [wbufs[b][pl.ds(i * 128 + o * L, L)]
                          for o in range(8)]
                    ga = [plsc.load_gather(dv, [r]) for r in rs]
                    gb = [plsc.load_gather(dv, [cl]) for cl in cs]
                    for o in range(8):
                        obufs[b][pl.ds(i * 128 + o * L, L)] = (
                            ga[o] * ws[o] * gb[o])
                    return carry2

                lax.fori_loop(0, B3, _inner, None)
                pltpu.async_copy(obufs[b], out_hbm.at[pl.ds(k * C3, C3)],
                                 osems[b])

        return carry

    lax.fori_loop(0, MX3 // 2, _pair, None)

    @pl.when(wid < NTAIL)
    def _tail():
        tbase = wid * TS

        def _it(i, carry):
            v = dv[pl.ds(tbase + i * L, L)]
            tbuf[pl.ds(i * L, L)] = v * v
            return carry

        lax.fori_loop(0, TS // L, _it, None, unroll=8)
        pltpu.sync_copy(tbuf, out_hbm.at[pl.ds(N_EDGES + tbase, TS)])

    for b in range(2):
        pltpu.make_async_copy(obufs[b], out_hbm.at[pl.ds(0, C3)],
                              osems[b]).wait()


_deg_call = pl.kernel(
    _deg_body,
    out_type=jax.ShapeDtypeStruct((NC * P,), jnp.float32),
    mesh=_mesh(),
    compiler_params=pltpu.CompilerParams(needs_layout_passes=False),
    scratch_types=[
        pltpu.VMEM((B1, 2, 128), jnp.int32),
        pltpu.VMEM((B1, 2, 128), jnp.int32),
        pltpu.VMEM((B1, 2, 128), jnp.int32),
        pltpu.VMEM((C1,), jnp.float32),
        pltpu.VMEM((C1,), jnp.float32),
        pltpu.VMEM((C1,), jnp.float32),
        pltpu.VMEM((C1,), jnp.int32),
        pltpu.VMEM((C1,), jnp.int32),
        pltpu.VMEM((C1,), jnp.int32),
        pltpu.VMEM((SLICE,), jnp.float32),
        pltpu.VMEM_SHARED((P,), jnp.float32),
        pltpu.SemaphoreType.DMA,
        pltpu.SemaphoreType.DMA,
        pltpu.SemaphoreType.DMA,
        pltpu.SemaphoreType.DMA,
        pltpu.SemaphoreType.DMA,
        pltpu.SemaphoreType.DMA,
    ],
)

_rsqrt_call = pl.kernel(
    _rsqrt_body,
    out_type=jax.ShapeDtypeStruct((P,), jnp.float32),
    mesh=_mesh(),
    compiler_params=pltpu.CompilerParams(needs_layout_passes=False),
    scratch_types=[
        pltpu.VMEM((WSLICE,), jnp.float32),
        pltpu.VMEM((WSLICE,), jnp.float32),
    ],
)

_norm_call = pl.kernel(
    _norm_body,
    out_type=jax.ShapeDtypeStruct((N_EDGES + N_NODES,), jnp.float32),
    mesh=_mesh(),
    compiler_params=pltpu.CompilerParams(needs_layout_passes=False),
    scratch_types=[
        pltpu.VMEM((P,), jnp.float32),
        pltpu.VMEM((B3, 2, 128), jnp.int32),
        pltpu.VMEM((B3, 2, 128), jnp.int32),
        pltpu.VMEM((C3,), jnp.float32),
        pltpu.VMEM((C3,), jnp.float32),
        pltpu.VMEM((C3,), jnp.float32),
        pltpu.VMEM((C3,), jnp.float32),
        pltpu.VMEM((TS,), jnp.float32),
        pltpu.SemaphoreType.DMA,
        pltpu.SemaphoreType.DMA,
        pltpu.SemaphoreType.DMA,
        pltpu.SemaphoreType.DMA,
        pltpu.SemaphoreType.DMA,
    ],
)


def kernel(edge_index, edge_weight):
    ei3 = edge_index.reshape(N_EDGES // 128, 128, 2).transpose(0, 2, 1)
    deg2 = _deg_call(ei3, edge_weight)
    dinv = _rsqrt_call(deg2)
    normed = _norm_call(ei3, edge_weight, dinv)
    ar = jnp.arange(N_NODES, dtype=edge_index.dtype)
    diag = jnp.stack([ar, ar], axis=1)
    ei = jnp.concatenate([edge_index, diag], axis=0)
    return ei, normed


# speedup vs baseline: 486.1466x; 1.0804x over previous
import functools

import jax
import jax.numpy as jnp
from jax import lax
from jax.experimental import pallas as pl
from jax.experimental.pallas import tpu as pltpu
from jax.experimental.pallas import tpu_sc as plsc

N_NODES = 100_000
N_EDGES = 6_400_000

NC = 2
NS = 16
NW = NC * NS
L = 16

P = 100_352
SLICE = P // NS
WSLICE = P // NW
NTAIL = 25
TS = N_NODES // NTAIL

C1 = 2_048
B1 = C1 // 128
NCH1 = N_EDGES // C1
MX1 = 98

C3 = 2_560
B3 = C3 // 128
NCH3 = N_EDGES // C3
MX3 = 80

assert NCH1 * C1 == N_EDGES and C1 % 128 == 0 and MX1 % 2 == 0
assert NCH3 * C3 == N_EDGES and C3 % 128 == 0 and MX3 % 2 == 0
assert MX1 >= -(-NCH1 // NW) and MX3 >= -(-NCH3 // NW)
assert P % (NS * L) == 0 and P % (NW * L) == 0 and P >= N_NODES
assert NTAIL * TS == N_NODES and TS % L == 0

_mesh = functools.partial(
    plsc.VectorSubcoreMesh,
    core_axis_name="c", subcore_axis_name="s", num_cores=NC, num_subcores=NS,
)


def _wid():
    return lax.axis_index("c") * NS + lax.axis_index("s")


def _deg_body(ei_hbm, ew_hbm, deg_hbm,
              dp, ebuf0, ebuf1, wbuf0, wbuf1, isem0, isem1):
    wid = _wid()
    ebufs, wbufs = (ebuf0, ebuf1), (wbuf0, wbuf1)
    isems = (isem0, isem1)

    def _issue_in(k, b):
        pltpu.async_copy(ei_hbm.at[pl.ds(k * B1, B1)], ebufs[b], isems[b])
        pltpu.async_copy(ew_hbm.at[pl.ds(k * C1, C1)], wbufs[b], isems[b])

    _issue_in(wid, 0)

    zeros16 = jnp.zeros((L,), jnp.float32)

    def _zero(i, carry):
        dp[pl.ds(i * L, L)] = zeros16
        return carry

    lax.fori_loop(0, P // L, _zero, None, unroll=8)

    def _pair(j2, carry):
        for b in range(2):
            j = j2 * 2 + b
            k = wid + j * NW
            knext = k + NW

            @pl.when(knext < NCH1)
            def _():
                _issue_in(knext, 1 - b)

            @pl.when(k < NCH1)
            def _():
                pltpu.make_async_copy(
                    ei_hbm.at[pl.ds(k * B1, B1)], ebufs[b], isems[b]).wait()
                pltpu.make_async_copy(
                    ew_hbm.at[pl.ds(k * C1, C1)], wbufs[b], isems[b]).wait()

                def _acc(i, carry2):
                    rs = [ebufs[b][i, 0, pl.ds(o * L, L)] for o in range(8)]
                    ws = [wbufs[b][pl.ds(i * 128 + o * L, L)]
                          for o in range(8)]
                    for o in range(8):
                        plsc.addupdate_scatter(dp, [rs[o]], ws[o])
                    return carry2

                lax.fori_loop(0, B1, _acc, None)

        return carry

    lax.fori_loop(0, MX1 // 2, _pair, None)
    pltpu.sync_copy(dp, deg_hbm.at[pl.ds(wid * P, P)])


def _rsqrt_body(deg_hbm, dinv_hbm, b0, b1, psem):
    base = _wid() * WSLICE
    for t in range(NW):
        pltpu.async_copy(deg_hbm.at[pl.ds(t * P + base, WSLICE)],
                         b1.at[pl.ds(t * WSLICE, WSLICE)], psem)
    for t in range(NW):
        pltpu.make_async_copy(deg_hbm.at[pl.ds(t * P + base, WSLICE)],
                              b1.at[pl.ds(t * WSLICE, WSLICE)], psem).wait()

    def _it(i, carry):
        sl = pl.ds(i * L, L)
        d = b1[pl.ds(i * L, L)] + 1.0
        for t in range(1, NW):
            d = d + b1[pl.ds(t * WSLICE + i * L, L)]
        bits = lax.bitcast_convert_type(d, jnp.int32)
        bits = 0x5F3759DF - lax.shift_right_arithmetic(bits, 1)
        y = lax.bitcast_convert_type(bits, jnp.float32)
        xh = d * 0.5
        y = y * (1.5 - xh * y * y)
        y = y * (1.5 - xh * y * y)
        y = y * (1.5 - xh * y * y)
        b0[sl] = y
        return carry

    lax.fori_loop(0, WSLICE // L, _it, None)
    pltpu.sync_copy(b0, dinv_hbm.at[pl.ds(base, WSLICE)])


def _norm_body(ei_hbm, ew_hbm, dinv_hbm, out_hbm,
               dv, ebuf0, ebuf1, wbuf0, wbuf1, obuf0, obuf1, tbuf,
               bsem, isem0, isem1, osem0, osem1):
    wid = _wid()
    ebufs, wbufs, obufs = (ebuf0, ebuf1), (wbuf0, wbuf1), (obuf0, obuf1)
    isems, osems = (isem0, isem1), (osem0, osem1)

    def _issue_in(k, b):
        pltpu.async_copy(ei_hbm.at[pl.ds(k * B3, B3)], ebufs[b], isems[b])
        pltpu.async_copy(ew_hbm.at[pl.ds(k * C3, C3)], wbufs[b], isems[b])

    _issue_in(wid, 0)

    for i in range(NW):
        p = (wid + i) % NW
        pltpu.async_copy(dinv_hbm.at[pl.ds(p * WSLICE, WSLICE)],
                         dv.at[pl.ds(p * WSLICE, WSLICE)], bsem)
    for i in range(NW):
        p = (wid + i) % NW
        pltpu.make_async_copy(dinv_hbm.at[pl.ds(p * WSLICE, WSLICE)],
                              dv.at[pl.ds(p * WSLICE, WSLICE)], bsem).wait()

    def _pair(j2, carry):
        for b in range(2):
            j = j2 * 2 + b
            k = wid + j * NW
            knext = k + NW

            @pl.when(knext < NCH3)
            def _():
                _issue_in(knext, 1 - b)

            @pl.when(k < NCH3)
            def _():
                pltpu.make_async_copy(
                    ei_hbm.at[pl.ds(k * B3, B3)], ebufs[b], isems[b]).wait()
                pltpu.make_async_copy(
                    ew_hbm.at[pl.ds(k * C3, C3)], wbufs[b], isems[b]).wait()

                @pl.when(j >= 2)
                def _():
                    pltpu.make_async_copy(
                        obufs[b], out_hbm.at[pl.ds(k * C3, C3)],
                        osems[b]).wait()

                def _inner(i, carry2):
                    rs = [ebufs[b][i, 0, pl.ds(o * L, L)] for o in range(8)]
                    cs = [ebufs[b][i, 1, pl.ds(o * L, L)] for o in range(8)]
                    ws = [wbufs[b][pl.ds(i * 128 + o * L, L)]
                          for o in range(8)]
                    ga = [plsc.load_gather(dv, [r]) for r in rs]
                    gb = [plsc.load_gather(dv, [cl]) for cl in cs]
                    for o in range(8):
                        obufs[b][pl.ds(i * 128 + o * L, L)] = (
                            ga[o] * ws[o] * gb[o])
                    return carry2

                lax.fori_loop(0, B3, _inner, None)
                pltpu.async_copy(obufs[b], out_hbm.at[pl.ds(k * C3, C3)],
                                 osems[b])

        return carry

    lax.fori_loop(0, MX3 // 2, _pair, None)

    @pl.when(wid < NTAIL)
    def _tail():
        tbase = wid * TS

        def _it(i, carry):
            v = dv[pl.ds(tbase + i * L, L)]
            tbuf[pl.ds(i * L, L)] = v * v
            return carry

        lax.fori_loop(0, TS // L, _it, None, unroll=8)
        pltpu.sync_copy(tbuf, out_hbm.at[pl.ds(N_EDGES + tbase, TS)])

    for b in range(2):
        pltpu.make_async_copy(obufs[b], out_hbm.at[pl.ds(0, C3)],
                              osems[b]).wait()


_deg_call = pl.kernel(
    _deg_body,
    out_type=jax.ShapeDtypeStruct((NW * P,), jnp.float32),
    mesh=_mesh(),
    compiler_params=pltpu.CompilerParams(needs_layout_passes=False),
    scratch_types=[
        pltpu.VMEM((P,), jnp.float32),
        pltpu.VMEM((B1, 2, 128), jnp.int32),
        pltpu.VMEM((B1, 2, 128), jnp.int32),
        pltpu.VMEM((C1,), jnp.float32),
        pltpu.VMEM((C1,), jnp.float32),
        pltpu.SemaphoreType.DMA,
        pltpu.SemaphoreType.DMA,
    ],
)

_rsqrt_call = pl.kernel(
    _rsqrt_body,
    out_type=jax.ShapeDtypeStruct((P,), jnp.float32),
    mesh=_mesh(),
    compiler_params=pltpu.CompilerParams(needs_layout_passes=False),
    scratch_types=[
        pltpu.VMEM((WSLICE,), jnp.float32),
        pltpu.VMEM((NW * WSLICE,), jnp.float32),
        pltpu.SemaphoreType.DMA,
    ],
)

_norm_call = pl.kernel(
    _norm_body,
    out_type=jax.ShapeDtypeStruct((N_EDGES + N_NODES,), jnp.float32),
    mesh=_mesh(),
    compiler_params=pltpu.CompilerParams(needs_layout_passes=False),
    scratch_types=[
        pltpu.VMEM((P,), jnp.float32),
        pltpu.VMEM((B3, 2, 128), jnp.int32),
        pltpu.VMEM((B3, 2, 128), jnp.int32),
        pltpu.VMEM((C3,), jnp.float32),
        pltpu.VMEM((C3,), jnp.float32),
        pltpu.VMEM((C3,), jnp.float32),
        pltpu.VMEM((C3,), jnp.float32),
        pltpu.VMEM((TS,), jnp.float32),
        pltpu.SemaphoreType.DMA,
        pltpu.SemaphoreType.DMA,
        pltpu.SemaphoreType.DMA,
        pltpu.SemaphoreType.DMA,
        pltpu.SemaphoreType.DMA,
    ],
)


def kernel(edge_index, edge_weight):
    ei3 = edge_index.reshape(N_EDGES // 128, 128, 2).transpose(0, 2, 1)
    deg2 = _deg_call(ei3, edge_weight)
    dinv = _rsqrt_call(deg2)
    normed = _norm_call(ei3, edge_weight, dinv)
    ar = jnp.arange(N_NODES, dtype=edge_index.dtype)
    diag = jnp.stack([ar, ar], axis=1)
    base = jnp.pad(edge_index, ((0, N_NODES), (0, 0)))
    ei = lax.dynamic_update_slice(base, diag, (N_EDGES, 0))
    return ei, normed


# speedup vs baseline: 498.4234x; 1.0253x over previous
import functools

import jax
import jax.numpy as jnp
from jax import lax
from jax.experimental import pallas as pl
from jax.experimental.pallas import tpu as pltpu
from jax.experimental.pallas import tpu_sc as plsc

N_NODES = 100_000
N_EDGES = 6_400_000

NC = 2
NS = 16
NW = NC * NS
L = 16

P = 100_352
SLICE = P // NS
WSLICE = P // NW
NTAIL = 25
TS = N_NODES // NTAIL

C1 = 2_048
B1 = C1 // 128
NCH1 = N_EDGES // C1
MX1 = 98

C3 = 2_560
B3 = C3 // 128
NCH3 = N_EDGES // C3
MX3 = 80

assert NCH1 * C1 == N_EDGES and C1 % 128 == 0 and MX1 % 2 == 0
assert NCH3 * C3 == N_EDGES and C3 % 128 == 0 and MX3 % 2 == 0
assert MX1 >= -(-NCH1 // NW) and MX3 >= -(-NCH3 // NW)
assert P % (NS * L) == 0 and P % (NW * L) == 0 and P >= N_NODES
assert NTAIL * TS == N_NODES and TS % L == 0

_mesh = functools.partial(
    plsc.VectorSubcoreMesh,
    core_axis_name="c", subcore_axis_name="s", num_cores=NC, num_subcores=NS,
)


def _wid():
    return lax.axis_index("c") * NS + lax.axis_index("s")


def _deg_body(ei_hbm, ew_hbm, deg_hbm,
              dp, ebuf0, ebuf1, wbuf0, wbuf1, isem0, isem1):
    wid = _wid()
    ebufs, wbufs = (ebuf0, ebuf1), (wbuf0, wbuf1)
    isems = (isem0, isem1)

    def _issue_in(k, b):
        pltpu.async_copy(ei_hbm.at[pl.ds(k * B1, B1), 0], ebufs[b], isems[b])
        pltpu.async_copy(ew_hbm.at[pl.ds(k * C1, C1)], wbufs[b], isems[b])

    _issue_in(wid, 0)

    zeros16 = jnp.zeros((L,), jnp.float32)

    def _zero(i, carry):
        dp[pl.ds(i * L, L)] = zeros16
        return carry

    lax.fori_loop(0, P // L, _zero, None, unroll=8)

    def _pair(j2, carry):
        for b in range(2):
            j = j2 * 2 + b
            k = wid + j * NW
            knext = k + NW

            @pl.when(knext < NCH1)
            def _():
                _issue_in(knext, 1 - b)

            @pl.when(k < NCH1)
            def _():
                pltpu.make_async_copy(
                    ei_hbm.at[pl.ds(k * B1, B1), 0], ebufs[b], isems[b]).wait()
                pltpu.make_async_copy(
                    ew_hbm.at[pl.ds(k * C1, C1)], wbufs[b], isems[b]).wait()

                def _acc(i, carry2):
                    rs = [ebufs[b][i, pl.ds(o * L, L)] for o in range(8)]
                    ws = [wbufs[b][pl.ds(i * 128 + o * L, L)]
                          for o in range(8)]
                    for o in range(8):
                        plsc.addupdate_scatter(dp, [rs[o]], ws[o])
                    return carry2

                lax.fori_loop(0, B1, _acc, None)

        return carry

    lax.fori_loop(0, MX1 // 2, _pair, None)
    pltpu.sync_copy(dp, deg_hbm.at[pl.ds(wid * P, P)])


def _rsqrt_body(deg_hbm, dinv_hbm, b0, b1, psem):
    base = _wid() * WSLICE
    for t in range(NW):
        pltpu.async_copy(deg_hbm.at[pl.ds(t * P + base, WSLICE)],
                         b1.at[pl.ds(t * WSLICE, WSLICE)], psem)
    for t in range(NW):
        pltpu.make_async_copy(deg_hbm.at[pl.ds(t * P + base, WSLICE)],
                              b1.at[pl.ds(t * WSLICE, WSLICE)], psem).wait()

    def _it(i, carry):
        sl = pl.ds(i * L, L)
        d = b1[pl.ds(i * L, L)] + 1.0
        for t in range(1, NW):
            d = d + b1[pl.ds(t * WSLICE + i * L, L)]
        bits = lax.bitcast_convert_type(d, jnp.int32)
        bits = 0x5F3759DF - lax.shift_right_arithmetic(bits, 1)
        y = lax.bitcast_convert_type(bits, jnp.float32)
        xh = d * 0.5
        y = y * (1.5 - xh * y * y)
        y = y * (1.5 - xh * y * y)
        y = y * (1.5 - xh * y * y)
        b0[sl] = y
        return carry

    lax.fori_loop(0, WSLICE // L, _it, None)
    pltpu.sync_copy(b0, dinv_hbm.at[pl.ds(base, WSLICE)])


def _norm_body(ei_hbm, ew_hbm, dinv_hbm, out_hbm,
               dv, ebuf0, ebuf1, wbuf0, wbuf1, obuf0, obuf1, tbuf,
               bsem, isem0, isem1, osem0, osem1):
    wid = _wid()
    ebufs, wbufs, obufs = (ebuf0, ebuf1), (wbuf0, wbuf1), (obuf0, obuf1)
    isems, osems = (isem0, isem1), (osem0, osem1)

    def _issue_in(k, b):
        pltpu.async_copy(ei_hbm.at[pl.ds(k * B3, B3)], ebufs[b], isems[b])
        pltpu.async_copy(ew_hbm.at[pl.ds(k * C3, C3)], wbufs[b], isems[b])

    _issue_in(wid, 0)

    for i in range(NW):
        p = (wid + i) % NW
        pltpu.async_copy(dinv_hbm.at[pl.ds(p * WSLICE, WSLICE)],
                         dv.at[pl.ds(p * WSLICE, WSLICE)], bsem)
    for i in range(NW):
        p = (wid + i) % NW
        pltpu.make_async_copy(dinv_hbm.at[pl.ds(p * WSLICE, WSLICE)],
                              dv.at[pl.ds(p * WSLICE, WSLICE)], bsem).wait()

    def _pair(j2, carry):
        for b in range(2):
            j = j2 * 2 + b
            k = wid + j * NW
            knext = k + NW

            @pl.when(knext < NCH3)
            def _():
                _issue_in(knext, 1 - b)

            @pl.when(k < NCH3)
            def _():
                pltpu.make_async_copy(
                    ei_hbm.at[pl.ds(k * B3, B3)], ebufs[b], isems[b]).wait()
                pltpu.make_async_copy(
                    ew_hbm.at[pl.ds(k * C3, C3)], wbufs[b], isems[b]).wait()

                @pl.when(j >= 2)
                def _():
                    pltpu.make_async_copy(
                        obufs[b], out_hbm.at[pl.ds(k * C3, C3)],
                        osems[b]).wait()

                def _inner(i, carry2):
                    rs = [ebufs[b][i, 0, pl.ds(o * L, L)] for o in range(8)]
                    cs = [ebufs[b][i, 1, pl.ds(o * L, L)] for o in range(8)]
                    ws = [wbufs[b][pl.ds(i * 128 + o * L, L)]
                          for o in range(8)]
                    ga = [plsc.load_gather(dv, [r]) for r in rs]
                    gb = [plsc.load_gather(dv, [cl]) for cl in cs]
                    for o in range(8):
                        obufs[b][pl.ds(i * 128 + o * L, L)] = (
                            ga[o] * ws[o] * gb[o])
                    return carry2

                lax.fori_loop(0, B3, _inner, None)
                pltpu.async_copy(obufs[b], out_hbm.at[pl.ds(k * C3, C3)],
                                 osems[b])

        return carry

    lax.fori_loop(0, MX3 // 2, _pair, None)

    @pl.when(wid < NTAIL)
    def _tail():
        tbase = wid * TS

        def _it(i, carry):
            v = dv[pl.ds(tbase + i * L, L)]
            tbuf[pl.ds(i * L, L)] = v * v
            return carry

        lax.fori_loop(0, TS // L, _it, None, unroll=8)
        pltpu.sync_copy(tbuf, out_hbm.at[pl.ds(N_EDGES + tbase, TS)])

    for b in range(2):
        pltpu.make_async_copy(obufs[b], out_hbm.at[pl.ds(0, C3)],
                              osems[b]).wait()


_deg_call = pl.kernel(
    _deg_body,
    out_type=jax.ShapeDtypeStruct((NW * P,), jnp.float32),
    mesh=_mesh(),
    compiler_params=pltpu.CompilerParams(needs_layout_passes=False),
    scratch_types=[
        pltpu.VMEM((P,), jnp.float32),
        pltpu.VMEM((B1, 128), jnp.int32),
        pltpu.VMEM((B1, 128), jnp.int32),
        pltpu.VMEM((C1,), jnp.float32),
        pltpu.VMEM((C1,), jnp.float32),
        pltpu.SemaphoreType.DMA,
        pltpu.SemaphoreType.DMA,
    ],
)

_rsqrt_call = pl.kernel(
    _rsqrt_body,
    out_type=jax.ShapeDtypeStruct((P,), jnp.float32),
    mesh=_mesh(),
    compiler_params=pltpu.CompilerParams(needs_layout_passes=False),
    scratch_types=[
        pltpu.VMEM((WSLICE,), jnp.float32),
        pltpu.VMEM((NW * WSLICE,), jnp.float32),
        pltpu.SemaphoreType.DMA,
    ],
)

_norm_call = pl.kernel(
    _norm_body,
    out_type=jax.ShapeDtypeStruct((N_EDGES + N_NODES,), jnp.float32),
    mesh=_mesh(),
    compiler_params=pltpu.CompilerParams(needs_layout_passes=False),
    scratch_types=[
        pltpu.VMEM((P,), jnp.float32),
        pltpu.VMEM((B3, 2, 128), jnp.int32),
        pltpu.VMEM((B3, 2, 128), jnp.int32),
        pltpu.VMEM((C3,), jnp.float32),
        pltpu.VMEM((C3,), jnp.float32),
        pltpu.VMEM((C3,), jnp.float32),
        pltpu.VMEM((C3,), jnp.float32),
        pltpu.VMEM((TS,), jnp.float32),
        pltpu.SemaphoreType.DMA,
        pltpu.SemaphoreType.DMA,
        pltpu.SemaphoreType.DMA,
        pltpu.SemaphoreType.DMA,
        pltpu.SemaphoreType.DMA,
    ],
)


def kernel(edge_index, edge_weight):
    ei3 = edge_index.reshape(N_EDGES // 128, 128, 2).transpose(0, 2, 1)
    deg2 = _deg_call(ei3, edge_weight)
    dinv = _rsqrt_call(deg2)
    normed = _norm_call(ei3, edge_weight, dinv)
    ar = jnp.arange(N_NODES, dtype=edge_index.dtype)
    diag = jnp.stack([ar, ar], axis=1)
    base = jnp.pad(edge_index, ((0, N_NODES), (0, 0)))
    ei = lax.dynamic_update_slice(base, diag, (N_EDGES, 0))
    return ei, normed


# speedup vs baseline: 580.7603x; 1.1652x over previous
import functools

import jax
import jax.numpy as jnp
from jax import lax
from jax.experimental import pallas as pl
from jax.experimental.pallas import tpu as pltpu
from jax.experimental.pallas import tpu_sc as plsc

N_NODES = 100_000
N_EDGES = 6_400_000

NC = 2
NS = 16
NW = NC * NS
L = 16

P = 100_352
SLICE = P // NS
WSLICE = P // NW
NTAIL = 25
TS = N_NODES // NTAIL

C1 = 2_048
B1 = C1 // 128
NCH1 = N_EDGES // C1
MX1 = 99

C3 = 2_048
B3 = C3 // 128
NCH3 = N_EDGES // C3
MX3 = 99

assert NCH1 * C1 == N_EDGES and C1 % 128 == 0 and MX1 % 3 == 0
assert NCH3 * C3 == N_EDGES and C3 % 128 == 0 and MX3 % 3 == 0
assert MX1 >= -(-NCH1 // NW) and MX3 >= -(-NCH3 // NW)
assert P % (NS * L) == 0 and P % (NW * L) == 0 and P >= N_NODES
assert NTAIL * TS == N_NODES and TS % L == 0

_mesh = functools.partial(
    plsc.VectorSubcoreMesh,
    core_axis_name="c", subcore_axis_name="s", num_cores=NC, num_subcores=NS,
)


def _wid():
    return lax.axis_index("c") * NS + lax.axis_index("s")


def _deg_body(ei_hbm, ew_hbm, deg_hbm,
              dp, ebuf0, ebuf1, ebuf2, wbuf0, wbuf1, wbuf2,
              isem0, isem1, isem2):
    wid = _wid()
    ebufs, wbufs = (ebuf0, ebuf1, ebuf2), (wbuf0, wbuf1, wbuf2)
    isems = (isem0, isem1, isem2)

    def _issue_in(k, b):
        pltpu.async_copy(ei_hbm.at[pl.ds(k * B1, B1), 0], ebufs[b], isems[b])
        pltpu.async_copy(ew_hbm.at[pl.ds(k * C1, C1)], wbufs[b], isems[b])

    _issue_in(wid, 0)
    _issue_in(wid + NW, 1)

    zeros16 = jnp.zeros((L,), jnp.float32)

    def _zero(i, carry):
        dp[pl.ds(i * L, L)] = zeros16
        return carry

    lax.fori_loop(0, P // L, _zero, None, unroll=8)

    def _trip(j3, carry):
        for b in range(3):
            j = j3 * 3 + b
            k = wid + j * NW

            @pl.when(k + 2 * NW < NCH1)
            def _():
                _issue_in(k + 2 * NW, (b + 2) % 3)

            @pl.when(k < NCH1)
            def _():
                pltpu.make_async_copy(
                    ei_hbm.at[pl.ds(k * B1, B1), 0], ebufs[b],
                    isems[b]).wait()
                pltpu.make_async_copy(
                    ew_hbm.at[pl.ds(k * C1, C1)], wbufs[b], isems[b]).wait()

                def _acc(i, carry2):
                    rs = [ebufs[b][i, pl.ds(o * L, L)] for o in range(8)]
                    ws = [wbufs[b][pl.ds(i * 128 + o * L, L)]
                          for o in range(8)]
                    for o in range(8):
                        plsc.addupdate_scatter(dp, [rs[o]], ws[o])
                    return carry2

                lax.fori_loop(0, B1, _acc, None)

        return carry

    lax.fori_loop(0, MX1 // 3, _trip, None)
    pltpu.sync_copy(dp, deg_hbm.at[pl.ds(wid * P, P)])


def _rsqrt_body(deg_hbm, dinv_hbm, b0, b1, psem):
    base = _wid() * WSLICE
    for t in range(NW):
        pltpu.async_copy(deg_hbm.at[pl.ds(t * P + base, WSLICE)],
                         b1.at[pl.ds(t * WSLICE, WSLICE)], psem)
    for t in range(NW):
        pltpu.make_async_copy(deg_hbm.at[pl.ds(t * P + base, WSLICE)],
                              b1.at[pl.ds(t * WSLICE, WSLICE)], psem).wait()

    def _it(i, carry):
        sl = pl.ds(i * L, L)
        d = b1[pl.ds(i * L, L)] + 1.0
        for t in range(1, NW):
            d = d + b1[pl.ds(t * WSLICE + i * L, L)]
        bits = lax.bitcast_convert_type(d, jnp.int32)
        bits = 0x5F3759DF - lax.shift_right_arithmetic(bits, 1)
        y = lax.bitcast_convert_type(bits, jnp.float32)
        xh = d * 0.5
        y = y * (1.5 - xh * y * y)
        y = y * (1.5 - xh * y * y)
        y = y * (1.5 - xh * y * y)
        b0[sl] = y
        return carry

    lax.fori_loop(0, WSLICE // L, _it, None)
    pltpu.sync_copy(b0, dinv_hbm.at[pl.ds(base, WSLICE)])


def _norm_body(ei_hbm, ew_hbm, dinv_hbm, out_hbm,
               dv, ebuf0, ebuf1, ebuf2, wbuf0, wbuf1, wbuf2,
               obuf0, obuf1, obuf2,
               bsem, isem0, isem1, isem2, osem0, osem1, osem2):
    wid = _wid()
    ebufs = (ebuf0, ebuf1, ebuf2)
    wbufs = (wbuf0, wbuf1, wbuf2)
    obufs = (obuf0, obuf1, obuf2)
    isems = (isem0, isem1, isem2)
    osems = (osem0, osem1, osem2)

    def _issue_in(k, b):
        pltpu.async_copy(ei_hbm.at[pl.ds(k * B3, B3)], ebufs[b], isems[b])
        pltpu.async_copy(ew_hbm.at[pl.ds(k * C3, C3)], wbufs[b], isems[b])

    _issue_in(wid, 0)
    _issue_in(wid + NW, 1)

    for i in range(NW):
        p = (wid + i) % NW
        pltpu.async_copy(dinv_hbm.at[pl.ds(p * WSLICE, WSLICE)],
                         dv.at[pl.ds(p * WSLICE, WSLICE)], bsem)
    for i in range(NW):
        p = (wid + i) % NW
        pltpu.make_async_copy(dinv_hbm.at[pl.ds(p * WSLICE, WSLICE)],
                              dv.at[pl.ds(p * WSLICE, WSLICE)], bsem).wait()

    def _trip(j3, carry):
        for b in range(3):
            j = j3 * 3 + b
            k = wid + j * NW

            @pl.when(k + 2 * NW < NCH3)
            def _():
                _issue_in(k + 2 * NW, (b + 2) % 3)

            @pl.when(k < NCH3)
            def _():
                pltpu.make_async_copy(
                    ei_hbm.at[pl.ds(k * B3, B3)], ebufs[b], isems[b]).wait()
                pltpu.make_async_copy(
                    ew_hbm.at[pl.ds(k * C3, C3)], wbufs[b], isems[b]).wait()

                @pl.when(j >= 3)
                def _():
                    pltpu.make_async_copy(
                        obufs[b], out_hbm.at[pl.ds(k * C3, C3)],
                        osems[b]).wait()

                def _inner(i, carry2):
                    rs = [ebufs[b][i, 0, pl.ds(o * L, L)] for o in range(8)]
                    cs = [ebufs[b][i, 1, pl.ds(o * L, L)] for o in range(8)]
                    ws = [wbufs[b][pl.ds(i * 128 + o * L, L)]
                          for o in range(8)]
                    ga = [plsc.load_gather(dv, [r]) for r in rs]
                    gb = [plsc.load_gather(dv, [cl]) for cl in cs]
                    for o in range(8):
                        obufs[b][pl.ds(i * 128 + o * L, L)] = (
                            ga[o] * ws[o] * gb[o])
                    return carry2

                lax.fori_loop(0, B3, _inner, None)
                pltpu.async_copy(obufs[b], out_hbm.at[pl.ds(k * C3, C3)],
                                 osems[b])

        return carry

    lax.fori_loop(0, MX3 // 3, _trip, None)

    for b in range(3):
        pltpu.make_async_copy(obufs[b], out_hbm.at[pl.ds(0, C3)],
                              osems[b]).wait()

    @pl.when(wid < NTAIL)
    def _tail():
        for h in range(2):
            tbase = wid * TS + h * (TS // 2)

            def _it(i, carry):
                v = dv[pl.ds(tbase + i * L, L)]
                obuf0[pl.ds(i * L, L)] = v * v
                return carry

            lax.fori_loop(0, TS // 2 // L, _it, None, unroll=5)
            pltpu.sync_copy(obuf0.at[pl.ds(0, TS // 2)],
                            out_hbm.at[pl.ds(N_EDGES + tbase, TS // 2)])


_deg_call = pl.kernel(
    _deg_body,
    out_type=jax.ShapeDtypeStruct((NW * P,), jnp.float32),
    mesh=_mesh(),
    compiler_params=pltpu.CompilerParams(needs_layout_passes=False),
    scratch_types=[
        pltpu.VMEM((P,), jnp.float32),
        pltpu.VMEM((B1, 128), jnp.int32),
        pltpu.VMEM((B1, 128), jnp.int32),
        pltpu.VMEM((B1, 128), jnp.int32),
        pltpu.VMEM((C1,), jnp.float32),
        pltpu.VMEM((C1,), jnp.float32),
        pltpu.VMEM((C1,), jnp.float32),
        pltpu.SemaphoreType.DMA,
        pltpu.SemaphoreType.DMA,
        pltpu.SemaphoreType.DMA,
    ],
)

_rsqrt_call = pl.kernel(
    _rsqrt_body,
    out_type=jax.ShapeDtypeStruct((P,), jnp.float32),
    mesh=_mesh(),
    compiler_params=pltpu.CompilerParams(needs_layout_passes=False),
    scratch_types=[
        pltpu.VMEM((WSLICE,), jnp.float32),
        pltpu.VMEM((NW * WSLICE,), jnp.float32),
        pltpu.SemaphoreType.DMA,
    ],
)

_norm_call = pl.kernel(
    _norm_body,
    out_type=jax.ShapeDtypeStruct((N_EDGES + N_NODES,), jnp.float32),
    mesh=_mesh(),
    compiler_params=pltpu.CompilerParams(needs_layout_passes=False),
    scratch_types=[
        pltpu.VMEM((P,), jnp.float32),
        pltpu.VMEM((B3, 2, 128), jnp.int32),
        pltpu.VMEM((B3, 2, 128), jnp.int32),
        pltpu.VMEM((B3, 2, 128), jnp.int32),
        pltpu.VMEM((C3,), jnp.float32),
        pltpu.VMEM((C3,), jnp.float32),
        pltpu.VMEM((C3,), jnp.float32),
        pltpu.VMEM((C3,), jnp.float32),
        pltpu.VMEM((C3,), jnp.float32),
        pltpu.VMEM((C3,), jnp.float32),
        pltpu.SemaphoreType.DMA,
        pltpu.SemaphoreType.DMA,
        pltpu.SemaphoreType.DMA,
        pltpu.SemaphoreType.DMA,
        pltpu.SemaphoreType.DMA,
        pltpu.SemaphoreType.DMA,
        pltpu.SemaphoreType.DMA,
    ],
)


def kernel(edge_index, edge_weight):
    ei3 = edge_index.reshape(N_EDGES // 128, 128, 2).transpose(0, 2, 1)
    deg2 = _deg_call(ei3, edge_weight)
    dinv = _rsqrt_call(deg2)
    normed = _norm_call(ei3, edge_weight, dinv)
    ar = jnp.arange(N_NODES, dtype=edge_index.dtype)
    diag = jnp.stack([ar, ar], axis=1)
    base = jnp.pad(edge_index, ((0, N_NODES), (0, 0)))
    ei = lax.dynamic_update_slice(base, diag, (N_EDGES, 0))
    return ei, normed


# speedup vs baseline: 623.0409x; 1.0728x over previous
import functools

import jax
import jax.numpy as jnp
from jax import lax
from jax.experimental import pallas as pl
from jax.experimental.pallas import tpu as pltpu
from jax.experimental.pallas import tpu_sc as plsc

N_NODES = 100_000
N_EDGES = 6_400_000

NC = 2
NS = 16
NW = NC * NS
L = 16

P = 100_352
SLICE = P // NS
WSLICE = P // NW
NTAIL = 25
TS = N_NODES // NTAIL

C1 = 3_200
B1 = C1 // 128
NCH1 = N_EDGES // C1
MX1 = 63

C3 = 2_048
B3 = C3 // 128
NCH3 = N_EDGES // C3
MX3 = 99

assert NCH1 * C1 == N_EDGES and C1 % 128 == 0 and MX1 % 3 == 0
assert NCH3 * C3 == N_EDGES and C3 % 128 == 0 and MX3 % 3 == 0
assert MX1 >= -(-NCH1 // NW) and MX3 >= -(-NCH3 // NW)
assert P % (NS * L) == 0 and P % (NW * L) == 0 and P >= N_NODES
assert NTAIL * TS == N_NODES and TS % L == 0

_mesh = functools.partial(
    plsc.VectorSubcoreMesh,
    core_axis_name="c", subcore_axis_name="s", num_cores=NC, num_subcores=NS,
)


def _wid():
    return lax.axis_index("c") * NS + lax.axis_index("s")


def _deg_body(ei_hbm, ew_hbm, deg_hbm,
              dp, ebuf0, ebuf1, ebuf2, wbuf0, wbuf1, wbuf2,
              isem0, isem1, isem2):
    wid = _wid()
    ebufs, wbufs = (ebuf0, ebuf1, ebuf2), (wbuf0, wbuf1, wbuf2)
    isems = (isem0, isem1, isem2)

    def _issue_in(k, b):
        pltpu.async_copy(ei_hbm.at[pl.ds(k * B1, B1), 0], ebufs[b], isems[b])
        pltpu.async_copy(ew_hbm.at[pl.ds(k * C1, C1)], wbufs[b], isems[b])

    _issue_in(wid, 0)
    _issue_in(wid + NW, 1)

    zeros16 = jnp.zeros((L,), jnp.float32)

    def _zero(i, carry):
        dp[pl.ds(i * L, L)] = zeros16
        return carry

    lax.fori_loop(0, P // L, _zero, None, unroll=8)

    def _trip(j3, carry):
        for b in range(3):
            j = j3 * 3 + b
            k = wid + j * NW

            @pl.when(k + 2 * NW < NCH1)
            def _():
                _issue_in(k + 2 * NW, (b + 2) % 3)

            @pl.when(k < NCH1)
            def _():
                pltpu.make_async_copy(
                    ei_hbm.at[pl.ds(k * B1, B1), 0], ebufs[b],
                    isems[b]).wait()
                pltpu.make_async_copy(
                    ew_hbm.at[pl.ds(k * C1, C1)], wbufs[b], isems[b]).wait()

                def _acc(i, carry2):
                    rs = [ebufs[b][i, pl.ds(o * L, L)] for o in range(8)]
                    ws = [wbufs[b][pl.ds(i * 128 + o * L, L)]
                          for o in range(8)]
                    for o in range(8):
                        plsc.addupdate_scatter(dp, [rs[o]], ws[o])
                    return carry2

                lax.fori_loop(0, B1, _acc, None)

        return carry

    lax.fori_loop(0, MX1 // 3, _trip, None)
    pltpu.sync_copy(dp, deg_hbm.at[pl.ds(wid * P, P)])


def _rsqrt_body(deg_hbm, dinv_hbm, b0, b1, psem):
    base = _wid() * WSLICE
    for t in range(NW):
        pltpu.async_copy(deg_hbm.at[pl.ds(t * P + base, WSLICE)],
                         b1.at[pl.ds(t * WSLICE, WSLICE)], psem)
    for t in range(NW):
        pltpu.make_async_copy(deg_hbm.at[pl.ds(t * P + base, WSLICE)],
                              b1.at[pl.ds(t * WSLICE, WSLICE)], psem).wait()

    def _it(i, carry):
        sl = pl.ds(i * L, L)
        d = b1[pl.ds(i * L, L)] + 1.0
        for t in range(1, NW):
            d = d + b1[pl.ds(t * WSLICE + i * L, L)]
        bits = lax.bitcast_convert_type(d, jnp.int32)
        bits = 0x5F3759DF - lax.shift_right_arithmetic(bits, 1)
        y = lax.bitcast_convert_type(bits, jnp.float32)
        xh = d * 0.5
        y = y * (1.5 - xh * y * y)
        y = y * (1.5 - xh * y * y)
        y = y * (1.5 - xh * y * y)
        b0[sl] = y
        return carry

    lax.fori_loop(0, WSLICE // L, _it, None, unroll=4)
    pltpu.sync_copy(b0, dinv_hbm.at[pl.ds(base, WSLICE)])


def _norm_body(ei_hbm, ew_hbm, dinv_hbm, out_hbm,
               dv, ebuf0, ebuf1, ebuf2, wbuf0, wbuf1, wbuf2,
               obuf0, obuf1, obuf2,
               bsem, isem0, isem1, isem2, osem0, osem1, osem2):
    wid = _wid()
    ebufs = (ebuf0, ebuf1, ebuf2)
    wbufs = (wbuf0, wbuf1, wbuf2)
    obufs = (obuf0, obuf1, obuf2)
    isems = (isem0, isem1, isem2)
    osems = (osem0, osem1, osem2)

    def _issue_in(k, b):
        pltpu.async_copy(ei_hbm.at[pl.ds(k * B3, B3)], ebufs[b], isems[b])
        pltpu.async_copy(ew_hbm.at[pl.ds(k * C3, C3)], wbufs[b], isems[b])

    _issue_in(wid, 0)
    _issue_in(wid + NW, 1)

    for i in range(NW):
        p = (wid + i) % NW
        pltpu.async_copy(dinv_hbm.at[pl.ds(p * WSLICE, WSLICE)],
                         dv.at[pl.ds(p * WSLICE, WSLICE)], bsem)
    for i in range(NW):
        p = (wid + i) % NW
        pltpu.make_async_copy(dinv_hbm.at[pl.ds(p * WSLICE, WSLICE)],
                              dv.at[pl.ds(p * WSLICE, WSLICE)], bsem).wait()

    def _trip(j3, carry):
        for b in range(3):
            j = j3 * 3 + b
            k = wid + j * NW

            @pl.when(k + 2 * NW < NCH3)
            def _():
                _issue_in(k + 2 * NW, (b + 2) % 3)

            @pl.when(k < NCH3)
            def _():
                pltpu.make_async_copy(
                    ei_hbm.at[pl.ds(k * B3, B3)], ebufs[b], isems[b]).wait()
                pltpu.make_async_copy(
                    ew_hbm.at[pl.ds(k * C3, C3)], wbufs[b], isems[b]).wait()

                @pl.when(j >= 3)
                def _():
                    pltpu.make_async_copy(
                        obufs[b], out_hbm.at[pl.ds(k * C3, C3)],
                        osems[b]).wait()

                def _inner(i, carry2):
                    rs = [ebufs[b][i, 0, pl.ds(o * L, L)] for o in range(8)]
                    cs = [ebufs[b][i, 1, pl.ds(o * L, L)] for o in range(8)]
                    ws = [wbufs[b][pl.ds(i * 128 + o * L, L)]
                          for o in range(8)]
                    ga = [plsc.load_gather(dv, [r]) for r in rs]
                    gb = [plsc.load_gather(dv, [cl]) for cl in cs]
                    for o in range(8):
                        obufs[b][pl.ds(i * 128 + o * L, L)] = (
                            ga[o] * ws[o] * gb[o])
                    return carry2

                lax.fori_loop(0, B3, _inner, None)
                pltpu.async_copy(obufs[b], out_hbm.at[pl.ds(k * C3, C3)],
                                 osems[b])

        return carry

    lax.fori_loop(0, MX3 // 3, _trip, None)

    for b in range(3):
        pltpu.make_async_copy(obufs[b], out_hbm.at[pl.ds(0, C3)],
                              osems[b]).wait()

    @pl.when(wid < NTAIL)
    def _tail():
        for h in range(2):
            tbase = wid * TS + h * (TS // 2)

            def _it(i, carry):
                v = dv[pl.ds(tbase + i * L, L)]
                obuf0[pl.ds(i * L, L)] = v * v
                return carry

            lax.fori_loop(0, TS // 2 // L, _it, None, unroll=5)
            pltpu.sync_copy(obuf0.at[pl.ds(0, TS // 2)],
                            out_hbm.at[pl.ds(N_EDGES + tbase, TS // 2)])


_deg_call = pl.kernel(
    _deg_body,
    out_type=jax.ShapeDtypeStruct((NW * P,), jnp.float32),
    mesh=_mesh(),
    compiler_params=pltpu.CompilerParams(needs_layout_passes=False),
    scratch_types=[
        pltpu.VMEM((P,), jnp.float32),
        pltpu.VMEM((B1, 128), jnp.int32),
        pltpu.VMEM((B1, 128), jnp.int32),
        pltpu.VMEM((B1, 128), jnp.int32),
        pltpu.VMEM((C1,), jnp.float32),
        pltpu.VMEM((C1,), jnp.float32),
        pltpu.VMEM((C1,), jnp.float32),
        pltpu.SemaphoreType.DMA,
        pltpu.SemaphoreType.DMA,
        pltpu.SemaphoreType.DMA,
    ],
)

_rsqrt_call = pl.kernel(
    _rsqrt_body,
    out_type=jax.ShapeDtypeStruct((P,), jnp.float32),
    mesh=_mesh(),
    compiler_params=pltpu.CompilerParams(needs_layout_passes=False),
    scratch_types=[
        pltpu.VMEM((WSLICE,), jnp.float32),
        pltpu.VMEM((NW * WSLICE,), jnp.float32),
        pltpu.SemaphoreType.DMA,
    ],
)

_norm_call = pl.kernel(
    _norm_body,
    out_type=jax.ShapeDtypeStruct((N_EDGES + N_NODES,), jnp.float32),
    mesh=_mesh(),
    compiler_params=pltpu.CompilerParams(needs_layout_passes=False),
    scratch_types=[
        pltpu.VMEM((P,), jnp.float32),
        pltpu.VMEM((B3, 2, 128), jnp.int32),
        pltpu.VMEM((B3, 2, 128), jnp.int32),
        pltpu.VMEM((B3, 2, 128), jnp.int32),
        pltpu.VMEM((C3,), jnp.float32),
        pltpu.VMEM((C3,), jnp.float32),
        pltpu.VMEM((C3,), jnp.float32),
        pltpu.VMEM((C3,), jnp.float32),
        pltpu.VMEM((C3,), jnp.float32),
        pltpu.VMEM((C3,), jnp.float32),
        pltpu.SemaphoreType.DMA,
        pltpu.SemaphoreType.DMA,
        pltpu.SemaphoreType.DMA,
        pltpu.SemaphoreType.DMA,
        pltpu.SemaphoreType.DMA,
        pltpu.SemaphoreType.DMA,
        pltpu.SemaphoreType.DMA,
    ],
)


def kernel(edge_index, edge_weight):
    ei3 = edge_index.reshape(N_EDGES // 128, 128, 2).transpose(0, 2, 1)
    deg2 = _deg_call(ei3, edge_weight)
    dinv = _rsqrt_call(deg2)
    normed = _norm_call(ei3, edge_weight, dinv)
    ar = jnp.arange(N_NODES, dtype=edge_index.dtype)
    diag = jnp.stack([ar, ar], axis=1)
    base = jnp.pad(edge_index, ((0, N_NODES), (0, 0)))
    ei = lax.dynamic_update_slice(base, diag, (N_EDGES, 0))
    return ei, normed


# speedup vs baseline: 638.0359x; 1.0241x over previous
import functools

import jax
import jax.numpy as jnp
from jax import lax
from jax.experimental import pallas as pl
from jax.experimental.pallas import tpu as pltpu
from jax.experimental.pallas import tpu_sc as plsc

N_NODES = 100_000
N_EDGES = 6_400_000

NC = 2
NS = 16
NW = NC * NS
L = 16

P = 100_352
SLICE = P // NS
WSLICE = P // NW
NTAIL = 25
TS = N_NODES // NTAIL

C1 = 3_200
B1 = C1 // 128
NCH1 = N_EDGES // C1
MX1 = 64

C3 = 2_560
B3 = C3 // 128
NCH3 = N_EDGES // C3
MX3 = 81
DV = 100_096

assert NCH1 * C1 == N_EDGES and C1 % 128 == 0 and MX1 % 4 == 0
assert NCH3 * C3 == N_EDGES and C3 % 128 == 0 and MX3 % 3 == 0
assert MX1 >= -(-NCH1 // NW) and MX3 >= -(-NCH3 // NW)
assert P % (NS * L) == 0 and P % (NW * L) == 0 and P >= N_NODES
assert NTAIL * TS == N_NODES and TS % L == 0

_mesh = functools.partial(
    plsc.VectorSubcoreMesh,
    core_axis_name="c", subcore_axis_name="s", num_cores=NC, num_subcores=NS,
)


def _wid():
    return lax.axis_index("c") * NS + lax.axis_index("s")


def _deg_body(ei_hbm, ew_hbm, deg_hbm,
              dp, ebuf0, ebuf1, ebuf2, ebuf3, wbuf0, wbuf1, wbuf2, wbuf3,
              isem0, isem1, isem2, isem3):
    wid = _wid()
    ebufs = (ebuf0, ebuf1, ebuf2, ebuf3)
    wbufs = (wbuf0, wbuf1, wbuf2, wbuf3)
    isems = (isem0, isem1, isem2, isem3)

    def _issue_in(k, b):
        pltpu.async_copy(ei_hbm.at[pl.ds(k * B1, B1), 0], ebufs[b], isems[b])
        pltpu.async_copy(ew_hbm.at[pl.ds(k * C1, C1)], wbufs[b], isems[b])

    for j0 in range(3):
        _issue_in(wid + j0 * NW, j0)

    zeros16 = jnp.zeros((L,), jnp.float32)

    def _zero(i, carry):
        dp[pl.ds(i * L, L)] = zeros16
        return carry

    lax.fori_loop(0, P // L, _zero, None, unroll=8)

    def _quad(j4, carry):
        for b in range(4):
            j = j4 * 4 + b
            k = wid + j * NW

            @pl.when(k + 3 * NW < NCH1)
            def _():
                _issue_in(k + 3 * NW, (b + 3) % 4)

            @pl.when(k < NCH1)
            def _():
                pltpu.make_async_copy(
                    ei_hbm.at[pl.ds(k * B1, B1), 0], ebufs[b],
                    isems[b]).wait()
                pltpu.make_async_copy(
                    ew_hbm.at[pl.ds(k * C1, C1)], wbufs[b], isems[b]).wait()

                def _acc(i, carry2):
                    rs = [ebufs[b][i, pl.ds(o * L, L)] for o in range(8)]
                    ws = [wbufs[b][pl.ds(i * 128 + o * L, L)]
                          for o in range(8)]
                    for o in range(8):
                        plsc.addupdate_scatter(dp, [rs[o]], ws[o])
                    return carry2

                lax.fori_loop(0, B1, _acc, None)

        return carry

    lax.fori_loop(0, MX1 // 4, _quad, None)
    pltpu.sync_copy(dp, deg_hbm.at[pl.ds(wid * P, P)])


def _rsqrt_body(deg_hbm, dinv_hbm, b0, b1, psem):
    base = _wid() * WSLICE
    for t in range(NW):
        pltpu.async_copy(deg_hbm.at[pl.ds(t * P + base, WSLICE)],
                         b1.at[pl.ds(t * WSLICE, WSLICE)], psem)
    for t in range(NW):
        pltpu.make_async_copy(deg_hbm.at[pl.ds(t * P + base, WSLICE)],
                              b1.at[pl.ds(t * WSLICE, WSLICE)], psem).wait()

    def _it(i, carry):
        sl = pl.ds(i * L, L)
        d = b1[pl.ds(i * L, L)] + 1.0
        for t in range(1, NW):
            d = d + b1[pl.ds(t * WSLICE + i * L, L)]
        bits = lax.bitcast_convert_type(d, jnp.int32)
        bits = 0x5F3759DF - lax.shift_right_arithmetic(bits, 1)
        y = lax.bitcast_convert_type(bits, jnp.float32)
        xh = d * 0.5
        y = y * (1.5 - xh * y * y)
        y = y * (1.5 - xh * y * y)
        y = y * (1.5 - xh * y * y)
        b0[sl] = y
        return carry

    lax.fori_loop(0, WSLICE // L, _it, None, unroll=4)
    pltpu.sync_copy(b0, dinv_hbm.at[pl.ds(base, WSLICE)])


def _norm_body(ei_hbm, ew_hbm, dinv_hbm, out_hbm,
               dv, ebuf0, ebuf1, ebuf2, wbuf0, wbuf1, wbuf2,
               obuf0, obuf1, obuf2,
               bsem, isem0, isem1, isem2, osem0, osem1, osem2):
    wid = _wid()
    ebufs = (ebuf0, ebuf1, ebuf2)
    wbufs = (wbuf0, wbuf1, wbuf2)
    obufs = (obuf0, obuf1, obuf2)
    isems = (isem0, isem1, isem2)
    osems = (osem0, osem1, osem2)

    def _issue_in(k, b):
        pltpu.async_copy(ei_hbm.at[pl.ds(k * B3, B3)], ebufs[b], isems[b])
        pltpu.async_copy(ew_hbm.at[pl.ds(k * C3, C3)], wbufs[b], isems[b])

    _issue_in(wid, 0)
    _issue_in(wid + NW, 1)

    BP = DV // NW
    def _bpiece(p):
        return (dinv_hbm.at[pl.ds(p * BP, BP)], dv.at[pl.ds(p * BP, BP)])
    for i in range(NW):
        s_, d_ = _bpiece((wid + i) % NW)
        pltpu.async_copy(s_, d_, bsem)
    for i in range(NW):
        s_, d_ = _bpiece((wid + i) % NW)
        pltpu.make_async_copy(s_, d_, bsem).wait()

    def _trip(j3, carry):
        for b in range(3):
            j = j3 * 3 + b
            k = wid + j * NW

            @pl.when(k + 2 * NW < NCH3)
            def _():
                _issue_in(k + 2 * NW, (b + 2) % 3)

            @pl.when(k < NCH3)
            def _():
                pltpu.make_async_copy(
                    ei_hbm.at[pl.ds(k * B3, B3)], ebufs[b], isems[b]).wait()
                pltpu.make_async_copy(
                    ew_hbm.at[pl.ds(k * C3, C3)], wbufs[b], isems[b]).wait()

                @pl.when(j >= 3)
                def _():
                    pltpu.make_async_copy(
                        obufs[b], out_hbm.at[pl.ds(k * C3, C3)],
                        osems[b]).wait()

                def _inner(i, carry2):
                    rs = [ebufs[b][i, 0, pl.ds(o * L, L)] for o in range(8)]
                    cs = [ebufs[b][i, 1, pl.ds(o * L, L)] for o in range(8)]
                    ws = [wbufs[b][pl.ds(i * 128 + o * L, L)]
                          for o in range(8)]
                    ga = [plsc.load_gather(dv, [r]) for r in rs]
                    gb = [plsc.load_gather(dv, [cl]) for cl in cs]
                    for o in range(8):
                        obufs[b][pl.ds(i * 128 + o * L, L)] = (
                            ga[o] * ws[o] * gb[o])
                    return carry2

                lax.fori_loop(0, B3, _inner, None)
                pltpu.async_copy(obufs[b], out_hbm.at[pl.ds(k * C3, C3)],
                                 osems[b])

        return carry

    lax.fori_loop(0, MX3 // 3, _trip, None)

    for b in range(3):
        pltpu.make_async_copy(obufs[b], out_hbm.at[pl.ds(0, C3)],
                              osems[b]).wait()

    @pl.when(wid < NTAIL)
    def _tail():
        for h in range(2):
            tbase = wid * TS + h * (TS // 2)

            def _it(i, carry):
                v = dv[pl.ds(tbase + i * L, L)]
                obuf0[pl.ds(i * L, L)] = v * v
                return carry

            lax.fori_loop(0, TS // 2 // L, _it, None, unroll=5)
            pltpu.sync_copy(obuf0.at[pl.ds(0, TS // 2)],
                            out_hbm.at[pl.ds(N_EDGES + tbase, TS // 2)])


_deg_call = pl.kernel(
    _deg_body,
    out_type=jax.ShapeDtypeStruct((NW * P,), jnp.float32),
    mesh=_mesh(),
    compiler_params=pltpu.CompilerParams(needs_layout_passes=False),
    scratch_types=[
        pltpu.VMEM((P,), jnp.float32),
        pltpu.VMEM((B1, 128), jnp.int32),
        pltpu.VMEM((B1, 128), jnp.int32),
        pltpu.VMEM((B1, 128), jnp.int32),
        pltpu.VMEM((B1, 128), jnp.int32),
        pltpu.VMEM((C1,), jnp.float32),
        pltpu.VMEM((C1,), jnp.float32),
        pltpu.VMEM((C1,), jnp.float32),
        pltpu.VMEM((C1,), jnp.float32),
        pltpu.SemaphoreType.DMA,
        pltpu.SemaphoreType.DMA,
        pltpu.SemaphoreType.DMA,
        pltpu.SemaphoreType.DMA,
    ],
)

_rsqrt_call = pl.kernel(
    _rsqrt_body,
    out_type=jax.ShapeDtypeStruct((P,), jnp.float32),
    mesh=_mesh(),
    compiler_params=pltpu.CompilerParams(needs_layout_passes=False),
    scratch_types=[
        pltpu.VMEM((WSLICE,), jnp.float32),
        pltpu.VMEM((NW * WSLICE,), jnp.float32),
        pltpu.SemaphoreType.DMA,
    ],
)

_norm_call = pl.kernel(
    _norm_body,
    out_type=jax.ShapeDtypeStruct((N_EDGES + N_NODES,), jnp.float32),
    mesh=_mesh(),
    compiler_params=pltpu.CompilerParams(needs_layout_passes=False),
    scratch_types=[
        pltpu.VMEM((DV,), jnp.float32),
        pltpu.VMEM((B3, 2, 128), jnp.int32),
        pltpu.VMEM((B3, 2, 128), jnp.int32),
        pltpu.VMEM((B3, 2, 128), jnp.int32),
        pltpu.VMEM((C3,), jnp.float32),
        pltpu.VMEM((C3,), jnp.float32),
        pltpu.VMEM((C3,), jnp.float32),
        pltpu.VMEM((C3,), jnp.float32),
        pltpu.VMEM((C3,), jnp.float32),
        pltpu.VMEM((C3,), jnp.float32),
        pltpu.SemaphoreType.DMA,
        pltpu.SemaphoreType.DMA,
        pltpu.SemaphoreType.DMA,
        pltpu.SemaphoreType.DMA,
        pltpu.SemaphoreType.DMA,
        pltpu.SemaphoreType.DMA,
        pltpu.SemaphoreType.DMA,
    ],
)


def kernel(edge_index, edge_weight):
    ei3 = edge_index.reshape(N_EDGES // 128, 128, 2).transpose(0, 2, 1)
    deg2 = _deg_call(ei3, edge_weight)
    dinv = _rsqrt_call(deg2)
    normed = _norm_call(ei3, edge_weight, dinv)
    ar = jnp.arange(N_NODES, dtype=edge_index.dtype)
    diag = jnp.stack([ar, ar], axis=1)
    base = jnp.pad(edge_index, ((0, N_NODES), (0, 0)))
    ei = lax.dynamic_update_slice(base, diag, (N_EDGES, 0))
    return ei, normed
